# Initial kernel scaffold; baseline (speedup 1.0000x reference)
import jax
import jax.numpy as jnp
from jax.experimental import pallas as pl


def kernel(pred_frac_eps_x, target_frac_eps_x, atom_batch, neighbor_direction, pred_edge_distance_score, lattice, batch_of_edge, symmetric_vector_noise):
    raise NotImplementedError("write your pallas kernel here")


# speedup vs baseline: 23.3004x; 23.3004x over previous
import functools

import jax
import jax.numpy as jnp
from jax import lax
from jax.experimental import pallas as pl
from jax.experimental.pallas import tpu as pltpu
from jax.experimental.pallas import tpu_sc as plsc

_NC = 2
_NS = 16
_NW = _NC * _NS
_L = 16

_CH_E = 2000
_CH_A = 1568
_EUNROLL = 5
_AUNROLL = 7

_NSWEEP = 8


def _sc_partials(pred, targ, abatch, ndir, score, ebatch, B):
    N = pred.shape[0]
    E = ndir.shape[0]
    KE = -(-E // (_NW * _CH_E))
    KA = -(-N // (_NW * _CH_A))
    EP = _NW * KE * _CH_E
    AP = _NW * KA * _CH_A
    if EP > E:
        ndir = jnp.pad(ndir, ((0, EP - E), (0, 0)))
        score = jnp.pad(score, (0, EP - E))
        ebatch = jnp.pad(ebatch, (0, EP - E))
    if AP > N:
        pred = jnp.pad(pred, ((0, AP - N), (0, 0)))
        targ = jnp.pad(targ, ((0, AP - N), (0, 0)))
        abatch = jnp.pad(abatch, (0, AP - N))
    ndir = jnp.reshape(ndir, (-1,))
    pred = jnp.reshape(pred, (-1,))
    targ = jnp.reshape(targ, (-1,))

    mesh = plsc.VectorSubcoreMesh(
        core_axis_name="c", subcore_axis_name="s",
        num_cores=_NC, num_subcores=_NS)

    @functools.partial(
        pl.kernel,
        out_type=(jax.ShapeDtypeStruct((_NW, 3 * B), jnp.float32),
                  jax.ShapeDtypeStruct((_NW, 2 * B), jnp.float32)),
        mesh=mesh,
        compiler_params=pltpu.CompilerParams(needs_layout_passes=False),
        scratch_types=[
            pltpu.VMEM((_L * 3 * B,), jnp.float32),
            pltpu.VMEM((_L * 2 * B,), jnp.float32),
            pltpu.VMEM((_CH_E * 3,), jnp.float32),
            pltpu.VMEM((_CH_E,), jnp.float32),
            pltpu.VMEM((_CH_E,), jnp.int32),
            pltpu.VMEM((_CH_A * 3,), jnp.float32),
            pltpu.VMEM((_CH_A * 3,), jnp.float32),
            pltpu.VMEM((_CH_A,), jnp.int32),
            pltpu.VMEM((3 * B,), jnp.float32),
            pltpu.VMEM((2 * B,), jnp.float32),
        ],
    )
    def sck(pred_h, targ_h, abatch_h, ndir_h, score_h, ebatch_h,
            oute_h, outa_h,
            acc_e, acc_a, edir_v, esc_v, eid_v, apred_v, atarg_v, aid_v,
            red_e, red_a):
        wid = lax.axis_index("s") * _NC + lax.axis_index("c")
        iota = lax.iota(jnp.int32, _L)
        iota3 = iota * 3
        zf = jnp.zeros((_L,), jnp.float32)
        onesf = zf + 1.0
        lane_e = iota * (3 * B)
        lane_a = iota * (2 * B)

        def zero_e(i, carry):
            acc_e[pl.ds(i * _L, _L)] = zf
            return carry
        lax.fori_loop(0, 3 * B, zero_e, None)

        def zero_a(i, carry):
            acc_a[pl.ds(i * _L, _L)] = zf
            return carry
        lax.fori_loop(0, 2 * B, zero_a, None)

        ebase0 = wid * (KE * _CH_E)

        def echunk(k, carry):
            base = ebase0 + k * _CH_E
            pltpu.sync_copy(ndir_h.at[pl.ds(base * 3, _CH_E * 3)], edir_v)
            pltpu.sync_copy(score_h.at[pl.ds(base, _CH_E)], esc_v)
            pltpu.sync_copy(ebatch_h.at[pl.ds(base, _CH_E)], eid_v)

            def estep(j, c2):
                for u in range(_EUNROLL):
                    o = (j * _EUNROLL + u) * _L
                    ids = eid_v[pl.ds(o, _L)]
                    sc = esc_v[pl.ds(o, _L)]
                    rows3 = o * 3 + iota3
                    dx = plsc.load_gather(edir_v, [rows3])
                    dy = plsc.load_gather(edir_v, [rows3 + 1])
                    dz = plsc.load_gather(edir_v, [rows3 + 2])
                    si = lane_e + ids
                    plsc.addupdate_scatter(acc_e, [si], sc * dx)
                    plsc.addupdate_scatter(acc_e, [si + B], sc * dy)
                    plsc.addupdate_scatter(acc_e, [si + 2 * B], sc * dz)
                return c2
            lax.fori_loop(0, _CH_E // (_L * _EUNROLL), estep, None)
            return carry
        lax.fori_loop(0, KE, echunk, None)

        abase0 = wid * (KA * _CH_A)

        def achunk(k, carry):
            base = abase0 + k * _CH_A
            pltpu.sync_copy(pred_h.at[pl.ds(base * 3, _CH_A * 3)], apred_v)
            pltpu.sync_copy(targ_h.at[pl.ds(base * 3, _CH_A * 3)], atarg_v)
            pltpu.sync_copy(abatch_h.at[pl.ds(base, _CH_A)], aid_v)

            def astep(j, c2):
                for u in range(_AUNROLL):
                    o = (j * _AUNROLL + u) * _L
                    ids = aid_v[pl.ds(o, _L)]
                    rows3 = o * 3 + iota3
                    sq = zf
                    for cc in range(3):
                        p = plsc.load_gather(apred_v, [rows3 + cc])
                        t = plsc.load_gather(atarg_v, [rows3 + cc])
                        d = jnp.abs(p - t)
                        d = jnp.minimum(d, 1.0)
                        w = jnp.minimum(d, 1.0 - d)
                        sq = sq + w * w
                    si = lane_a + ids
                    plsc.addupdate_scatter(acc_a, [si], sq)
                    gmask = (base + o + iota) < N
                    plsc.addupdate_scatter(acc_a, [si + B], onesf, mask=gmask)
                return c2
            lax.fori_loop(0, _CH_A // (_L * _AUNROLL), astep, None)
            return carry
        lax.fori_loop(0, KA, achunk, None)

        def rede(v, carry):
            o = v * _L
            s = acc_e[pl.ds(o, _L)]
            for l in range(1, _L):
                s = s + acc_e[pl.ds(l * 3 * B + o, _L)]
            red_e[pl.ds(o, _L)] = s
            return carry
        lax.fori_loop(0, (3 * B) // _L, rede, None)

        def reda(v, carry):
            o = v * _L
            s = acc_a[pl.ds(o, _L)]
            for l in range(1, _L):
                s = s + acc_a[pl.ds(l * 2 * B + o, _L)]
            red_a[pl.ds(o, _L)] = s
            return carry
        lax.fori_loop(0, (2 * B) // _L, reda, None)

        pltpu.sync_copy(red_e, oute_h.at[wid])
        pltpu.sync_copy(red_a, outa_h.at[wid])

    return sck(pred, targ, abatch, ndir, score, ebatch)


def _combine_body(oute, outa, lat, noise, out):
    B = noise.shape[1]
    e = oute[...]
    S = [jnp.sum(e[:, i * B:(i + 1) * B], axis=0) for i in range(3)]
    a = outa[...]
    seg_sum = jnp.sum(a[:, 0:B], axis=0)
    seg_cnt = jnp.sum(a[:, B:2 * B], axis=0)
    err_x = seg_sum / jnp.maximum(seg_cnt, 1.0)
    mean_err_x = jnp.sum(err_x) / B

    m = [[lat[3 * i + j, :] * S[i] for j in range(3)] for i in range(3)]
    A = [[None] * 3 for _ in range(3)]
    for j in range(3):
        for k in range(j, 3):
            A[j][k] = m[0][j] * m[0][k] + m[1][j] * m[1][k] + m[2][j] * m[2][k]
            A[k][j] = A[j][k]
    one = jnp.ones((B,), jnp.float32)
    zero = jnp.zeros((B,), jnp.float32)
    V = [[one if i == j else zero for j in range(3)] for i in range(3)]

    for _ in range(_NSWEEP):
        for (p, q) in ((0, 1), (0, 2), (1, 2)):
            app, aqq, apq = A[p][p], A[q][q], A[p][q]
            nz = apq != 0.0
            apq_s = jnp.where(nz, apq, 1.0)
            tau = (aqq - app) / (2.0 * apq_s)
            sgn = jnp.where(tau >= 0.0, 1.0, -1.0)
            t = sgn / (jnp.abs(tau) + jnp.sqrt(1.0 + tau * tau))
            t = jnp.where(nz, t, 0.0)
            c = 1.0 / jnp.sqrt(1.0 + t * t)
            s = t * c
            r = 3 - p - q
            arp = c * A[r][p] - s * A[r][q]
            arq = s * A[r][p] + c * A[r][q]
            A[p][p] = app - t * apq
            A[q][q] = aqq + t * apq
            A[p][q] = zero
            A[q][p] = zero
            A[r][p] = arp
            A[p][r] = arp
            A[r][q] = arq
            A[q][r] = arq
            for i in range(3):
                vip = c * V[i][p] - s * V[i][q]
                viq = s * V[i][p] + c * V[i][q]
                V[i][p] = vip
                V[i][q] = viq

    rt = [jnp.sqrt(jnp.maximum(A[k][k], 0.0)) for k in range(3)]

    def sym(i, j):
        return (V[i][0] * rt[0] * V[j][0]
                + V[i][1] * rt[1] * V[j][1]
                + V[i][2] * rt[2] * V[j][2])

    comps = (sym(0, 0), sym(1, 1), sym(2, 2), sym(0, 1), sym(0, 2), sym(1, 2))
    tot = jnp.zeros((), jnp.float32)
    for ci in range(6):
        d = comps[ci] - noise[ci, :]
        tot = tot + jnp.sum(d * d)
    err_l = tot / (6.0 * B)
    out[0, 0] = mean_err_x + err_l


def _tc_combine(oute, outa, lat_t, noise_t, interpret=False):
    res = pl.pallas_call(
        _combine_body,
        out_shape=jax.ShapeDtypeStruct((1, 1), jnp.float32),
        out_specs=pl.BlockSpec(memory_space=pltpu.SMEM),
        interpret=interpret,
    )(oute, outa, lat_t, noise_t)
    return res[0, 0]


def kernel(pred_frac_eps_x, target_frac_eps_x, atom_batch, neighbor_direction,
           pred_edge_distance_score, lattice, batch_of_edge,
           symmetric_vector_noise):
    B = lattice.shape[0]
    pred = pred_frac_eps_x.astype(jnp.float32)
    targ = target_frac_eps_x.astype(jnp.float32)
    ab = atom_batch.astype(jnp.int32)
    nd = neighbor_direction.astype(jnp.float32)
    sc = jnp.reshape(pred_edge_distance_score, (-1,)).astype(jnp.float32)
    eb = batch_of_edge.astype(jnp.int32)
    oute, outa = _sc_partials(pred, targ, ab, nd, sc, eb, B)
    lat_t = jnp.transpose(jnp.reshape(lattice.astype(jnp.float32), (B, 9)))
    noise_t = jnp.transpose(symmetric_vector_noise.astype(jnp.float32))
    return _tc_combine(oute, outa, lat_t, noise_t)


# speedup vs baseline: 453.9684x; 19.4833x over previous
import functools

import jax
import jax.numpy as jnp
from jax import lax
from jax.experimental import pallas as pl
from jax.experimental.pallas import tpu as pltpu
from jax.experimental.pallas import tpu_sc as plsc

_NC = 2
_NS = 16
_NW = _NC * _NS
_L = 16

_CH_E = 2000
_CH_A = 1568
_EUNROLL = 5
_AUNROLL = 7

_NSWEEP = 8


def _sc_partials(px, py, pz, tx, ty, tz, abatch, dx, dy, dz, score, ebatch, B):
    N = px.shape[0]
    E = dx.shape[0]
    KE = -(-E // (_NW * _CH_E))
    KA = -(-N // (_NW * _CH_A))
    EP = _NW * KE * _CH_E
    AP = _NW * KA * _CH_A
    if EP > E:
        dx, dy, dz, score, ebatch = (
            jnp.pad(a, (0, EP - E)) for a in (dx, dy, dz, score, ebatch))
    if AP > N:
        px, py, pz, tx, ty, tz, abatch = (
            jnp.pad(a, (0, AP - N)) for a in (px, py, pz, tx, ty, tz, abatch))

    ESTRIDE = 3 * B + 1
    ASTRIDE = 2 * B + 1

    mesh = plsc.VectorSubcoreMesh(
        core_axis_name="c", subcore_axis_name="s",
        num_cores=_NC, num_subcores=_NS)

    @functools.partial(
        pl.kernel,
        out_type=(jax.ShapeDtypeStruct((_NW, 3 * B), jnp.float32),
                  jax.ShapeDtypeStruct((_NW, 2 * B), jnp.float32)),
        mesh=mesh,
        compiler_params=pltpu.CompilerParams(needs_layout_passes=False),
        scratch_types=[
            pltpu.VMEM((_L * ESTRIDE,), jnp.float32),
            pltpu.VMEM((_L * ASTRIDE,), jnp.float32),
            pltpu.VMEM((_CH_E,), jnp.float32),
            pltpu.VMEM((_CH_E,), jnp.float32),
            pltpu.VMEM((_CH_E,), jnp.float32),
            pltpu.VMEM((_CH_E,), jnp.float32),
            pltpu.VMEM((_CH_E,), jnp.int32),
            pltpu.VMEM((_CH_A,), jnp.float32),
            pltpu.VMEM((_CH_A,), jnp.float32),
            pltpu.VMEM((_CH_A,), jnp.float32),
            pltpu.VMEM((_CH_A,), jnp.float32),
            pltpu.VMEM((_CH_A,), jnp.float32),
            pltpu.VMEM((_CH_A,), jnp.float32),
            pltpu.VMEM((_CH_A,), jnp.int32),
            pltpu.VMEM((3 * B,), jnp.float32),
            pltpu.VMEM((2 * B,), jnp.float32),
            pltpu.SemaphoreType.DMA,
        ],
    )
    def sck(px_h, py_h, pz_h, tx_h, ty_h, tz_h, abatch_h,
            dx_h, dy_h, dz_h, score_h, ebatch_h,
            oute_h, outa_h,
            acc_e, acc_a, dx_v, dy_v, dz_v, esc_v, eid_v,
            px_v, py_v, pz_v, tx_v, ty_v, tz_v, aid_v,
            red_e, red_a, sem):
        wid = lax.axis_index("s") * _NC + lax.axis_index("c")
        iota = lax.iota(jnp.int32, _L)
        zf = jnp.zeros((_L,), jnp.float32)
        onesf = zf + 1.0
        lane_e = iota * ESTRIDE
        lane_a = iota * ASTRIDE

        def zero_e(i, carry):
            acc_e[pl.ds(i * _L, _L)] = zf
            return carry
        lax.fori_loop(0, _L * ESTRIDE // _L, zero_e, None)

        def zero_a(i, carry):
            acc_a[pl.ds(i * _L, _L)] = zf
            return carry
        lax.fori_loop(0, _L * ASTRIDE // _L, zero_a, None)

        ebase0 = wid * (KE * _CH_E)

        def echunk(k, carry):
            base = ebase0 + k * _CH_E
            cps = ((dx_h, dx_v), (dy_h, dy_v), (dz_h, dz_v),
                   (score_h, esc_v), (ebatch_h, eid_v))
            for src, dst in cps:
                pltpu.async_copy(src.at[pl.ds(base, _CH_E)], dst, sem)
            for src, dst in cps:
                pltpu.make_async_copy(src.at[pl.ds(base, _CH_E)], dst, sem).wait()

            def estep(j, c2):
                for u in range(_EUNROLL):
                    o = (j * _EUNROLL + u) * _L
                    sl = pl.ds(o, _L)
                    ids = eid_v[sl]
                    sc = esc_v[sl]
                    si = lane_e + ids
                    plsc.addupdate_scatter(acc_e, [si], sc * dx_v[sl])
                    plsc.addupdate_scatter(acc_e, [si + B], sc * dy_v[sl])
                    plsc.addupdate_scatter(acc_e, [si + 2 * B], sc * dz_v[sl])
                return c2
            lax.fori_loop(0, _CH_E // (_L * _EUNROLL), estep, None)
            return carry
        lax.fori_loop(0, KE, echunk, None)

        abase0 = wid * (KA * _CH_A)

        def achunk(k, carry):
            base = abase0 + k * _CH_A
            cps = ((px_h, px_v), (py_h, py_v), (pz_h, pz_v),
                   (tx_h, tx_v), (ty_h, ty_v), (tz_h, tz_v),
                   (abatch_h, aid_v))
            for src, dst in cps:
                pltpu.async_copy(src.at[pl.ds(base, _CH_A)], dst, sem)
            for src, dst in cps:
                pltpu.make_async_copy(src.at[pl.ds(base, _CH_A)], dst, sem).wait()

            def astep(j, c2):
                for u in range(_AUNROLL):
                    o = (j * _AUNROLL + u) * _L
                    sl = pl.ds(o, _L)
                    ids = aid_v[sl]
                    sq = zf
                    for pv, tv in ((px_v, tx_v), (py_v, ty_v), (pz_v, tz_v)):
                        d = jnp.abs(pv[sl] - tv[sl])
                        d = jnp.minimum(d, 1.0)
                        w = jnp.minimum(d, 1.0 - d)
                        sq = sq + w * w
                    si = lane_a + ids
                    plsc.addupdate_scatter(acc_a, [si], sq)
                    gmask = (base + o + iota) < N
                    plsc.addupdate_scatter(acc_a, [si + B], onesf, mask=gmask)
                return c2
            lax.fori_loop(0, _CH_A // (_L * _AUNROLL), astep, None)
            return carry
        lax.fori_loop(0, KA, achunk, None)

        def rede(v, carry):
            o = v * _L
            s = acc_e[pl.ds(o, _L)]
            for l in range(1, _L):
                s = s + acc_e[pl.ds(l * ESTRIDE + o, _L)]
            red_e[pl.ds(o, _L)] = s
            return carry
        lax.fori_loop(0, (3 * B) // _L, rede, None)

        def reda(v, carry):
            o = v * _L
            s = acc_a[pl.ds(o, _L)]
            for l in range(1, _L):
                s = s + acc_a[pl.ds(l * ASTRIDE + o, _L)]
            red_a[pl.ds(o, _L)] = s
            return carry
        lax.fori_loop(0, (2 * B) // _L, reda, None)

        pltpu.sync_copy(red_e, oute_h.at[wid])
        pltpu.sync_copy(red_a, outa_h.at[wid])

    return sck(px, py, pz, tx, ty, tz, abatch, dx, dy, dz, score, ebatch)


def _combine_body(oute, outa, lat, noise, out):
    B = noise.shape[1]
    e = oute[...]
    S = [jnp.sum(e[:, i * B:(i + 1) * B], axis=0) for i in range(3)]
    a = outa[...]
    seg_sum = jnp.sum(a[:, 0:B], axis=0)
    seg_cnt = jnp.sum(a[:, B:2 * B], axis=0)
    err_x = seg_sum / jnp.maximum(seg_cnt, 1.0)
    mean_err_x = jnp.sum(err_x) / B

    m = [[lat[3 * i + j, :] * S[i] for j in range(3)] for i in range(3)]
    A = [[None] * 3 for _ in range(3)]
    for j in range(3):
        for k in range(j, 3):
            A[j][k] = m[0][j] * m[0][k] + m[1][j] * m[1][k] + m[2][j] * m[2][k]
            A[k][j] = A[j][k]
    one = jnp.ones((B,), jnp.float32)
    zero = jnp.zeros((B,), jnp.float32)
    V = [[one if i == j else zero for j in range(3)] for i in range(3)]

    for _ in range(_NSWEEP):
        for (p, q) in ((0, 1), (0, 2), (1, 2)):
            app, aqq, apq = A[p][p], A[q][q], A[p][q]
            nz = apq != 0.0
            apq_s = jnp.where(nz, apq, 1.0)
            tau = (aqq - app) / (2.0 * apq_s)
            sgn = jnp.where(tau >= 0.0, 1.0, -1.0)
            t = sgn / (jnp.abs(tau) + jnp.sqrt(1.0 + tau * tau))
            t = jnp.where(nz, t, 0.0)
            c = 1.0 / jnp.sqrt(1.0 + t * t)
            s = t * c
            r = 3 - p - q
            arp = c * A[r][p] - s * A[r][q]
            arq = s * A[r][p] + c * A[r][q]
            A[p][p] = app - t * apq
            A[q][q] = aqq + t * apq
            A[p][q] = zero
            A[q][p] = zero
            A[r][p] = arp
            A[p][r] = arp
            A[r][q] = arq
            A[q][r] = arq
            for i in range(3):
                vip = c * V[i][p] - s * V[i][q]
                viq = s * V[i][p] + c * V[i][q]
                V[i][p] = vip
                V[i][q] = viq

    rt = [jnp.sqrt(jnp.maximum(A[k][k], 0.0)) for k in range(3)]

    def sym(i, j):
        return (V[i][0] * rt[0] * V[j][0]
                + V[i][1] * rt[1] * V[j][1]
                + V[i][2] * rt[2] * V[j][2])

    comps = (sym(0, 0), sym(1, 1), sym(2, 2), sym(0, 1), sym(0, 2), sym(1, 2))
    tot = jnp.zeros((), jnp.float32)
    for ci in range(6):
        d = comps[ci] - noise[ci, :]
        tot = tot + jnp.sum(d * d)
    err_l = tot / (6.0 * B)
    out[0, 0] = mean_err_x + err_l


def _tc_combine(oute, outa, lat_t, noise_t, interpret=False):
    res = pl.pallas_call(
        _combine_body,
        out_shape=jax.ShapeDtypeStruct((1, 1), jnp.float32),
        out_specs=pl.BlockSpec(memory_space=pltpu.SMEM),
        interpret=interpret,
    )(oute, outa, lat_t, noise_t)
    return res[0, 0]


def kernel(pred_frac_eps_x, target_frac_eps_x, atom_batch, neighbor_direction,
           pred_edge_distance_score, lattice, batch_of_edge,
           symmetric_vector_noise):
    B = lattice.shape[0]
    pred = pred_frac_eps_x.astype(jnp.float32)
    targ = target_frac_eps_x.astype(jnp.float32)
    nd = neighbor_direction.astype(jnp.float32)
    ab = atom_batch.astype(jnp.int32)
    sc = jnp.reshape(pred_edge_distance_score, (-1,)).astype(jnp.float32)
    eb = batch_of_edge.astype(jnp.int32)
    oute, outa = _sc_partials(
        pred[:, 0], pred[:, 1], pred[:, 2],
        targ[:, 0], targ[:, 1], targ[:, 2], ab,
        nd[:, 0], nd[:, 1], nd[:, 2], sc, eb, B)
    lat_t = jnp.transpose(jnp.reshape(lattice.astype(jnp.float32), (B, 9)))
    noise_t = jnp.transpose(symmetric_vector_noise.astype(jnp.float32))
    return _tc_combine(oute, outa, lat_t, noise_t)
